# baseline (speedup 1.0000x reference)
import functools

import jax
import jax.numpy as jnp
from jax import lax
from jax.experimental import pallas as pl
from jax.experimental.pallas import tpu as pltpu
from jax.experimental.pallas import tpu_sc as plsc

VOCAB = 1000000
DIM = 32
BATCH = 16384
HIST = 50

NC = 2
NS = 16
NW = NC * NS
RPW = BATCH // NW
CB = 16
NCHUNK = RPW // CB

_mesh = plsc.VectorSubcoreMesh(core_axis_name="c", subcore_axis_name="s")


@functools.partial(
    pl.kernel,
    mesh=_mesh,
    out_type=jax.ShapeDtypeStruct((BATCH, HIST, DIM), jnp.float32),
    scratch_types=[
        pltpu.VMEM((2, CB, HIST), jnp.int32),
        pltpu.VMEM((2, CB, HIST, DIM), jnp.float32),
        pltpu.SemaphoreType.DMA,
        pltpu.SemaphoreType.DMA,
        pltpu.SemaphoreType.DMA,
        pltpu.SemaphoreType.DMA,
        pltpu.SemaphoreType.DMA,
        pltpu.SemaphoreType.DMA,
    ],
    compiler_params=pltpu.CompilerParams(use_tc_tiling_on_sc=False),
)
def _emb_gather(
    data_hbm, table_hbm, out_hbm, idx_v, rows_v, si0, si1, sg0, sg1, so0, so1
):
    wid = lax.axis_index("s") * NC + lax.axis_index("c")
    row0 = wid * RPW
    si = (si0, si1)
    sg = (sg0, sg1)
    so = (so0, so1)

    def stage_idx(g):
        return pltpu.async_copy(
            data_hbm.at[pl.ds(row0 + g * CB, CB)], idx_v.at[g % 2], si[g % 2]
        )

    def gather(g):
        return [
            pltpu.async_copy(
                table_hbm.at[idx_v.at[g % 2, i]],
                rows_v.at[g % 2, i],
                sg[g % 2],
            )
            for i in range(CB)
        ]

    def put(g):
        return pltpu.async_copy(
            rows_v.at[g % 2], out_hbm.at[pl.ds(row0 + g * CB, CB)], so[g % 2]
        )

    h_i = [None, None]
    h_g = [None, None]
    h_o = [None, None]
    h_i[0] = stage_idx(0)
    h_i[0].wait()
    h_g[0] = gather(0)
    h_i[1] = stage_idx(1)
    for g in range(NCHUNK):
        if g + 1 < NCHUNK:
            if g >= 1:
                h_o[(g + 1) % 2].wait()
            h_i[(g + 1) % 2].wait()
            h_g[(g + 1) % 2] = gather(g + 1)
        for h in h_g[g % 2]:
            h.wait()
        if g + 2 < NCHUNK:
            h_i[g % 2] = stage_idx(g + 2)
        h_o[g % 2] = put(g)
    h_o[0].wait()
    h_o[1].wait()


def kernel(data, weight):
    return _emb_gather(data, weight)
